# baseline (speedup 1.0000x reference)
import functools
import jax
import jax.numpy as jnp
from jax import lax
from jax.experimental import pallas as pl
from jax.experimental.pallas import tpu as pltpu

N_PTS = 16384
N_IN = 256
N_OUT = 256
B = 4
R = 512
NBLK = N_PTS // R
SEG = (N_PTS // B) * 8


def _sortable_i32(f32):
    i = lax.bitcast_convert_type(f32, jnp.int32)
    return i ^ (lax.shift_right_arithmetic(i, 31) & jnp.int32(0x7FFFFFFF))


def _scores_body(x_ref, wupf_ref, bupt_ref, w1_ref, b1_ref, w2_ref, b2_ref,
                 s_ref):
    xb = x_ref[...]
    up_all = jnp.dot(xb, wupf_ref[...], preferred_element_type=jnp.float32)
    up_all = up_all + bupt_ref[...]
    for o in range(8):
        up_o = up_all[:, 256 * o:256 * (o + 1)].astype(jnp.bfloat16)
        h = jnp.dot(up_o, w1_ref[...], preferred_element_type=jnp.float32)
        h = jnp.maximum(h + b1_ref[...], 0.0).astype(jnp.bfloat16)
        s = jnp.dot(h, w2_ref[...], preferred_element_type=jnp.float32)
        s_ref[:, o:o + 1] = s[:, 0:1] + b2_ref[0, 0]


def _thresh_body(s2_ref, k_ref, thr_ref):
    i32 = _sortable_i32(s2_ref[...])
    ukey = lax.bitcast_convert_type(i32, jnp.uint32) ^ jnp.uint32(0x80000000)
    rows = SEG // 128
    idx = (lax.broadcasted_iota(jnp.int32, (rows, 128), 0) * 128
           + lax.broadcasted_iota(jnp.int32, (rows, 128), 1))
    ones = jnp.ones((128, 128), jnp.float32)

    def total(pred_f32):
        part = jnp.sum(pred_f32, axis=0, keepdims=True)
        return jnp.dot(part, ones, preferred_element_type=jnp.float32)

    for b in range(B):
        useg = ukey[b * rows:(b + 1) * rows, :]
        kb = (k_ref[b]).astype(jnp.float32)

        def tbody(j, t):
            bit = jnp.uint32(31) - j.astype(jnp.uint32)
            tt = t | (jnp.uint32(1) << bit)
            cnt = total((useg >= tt).astype(jnp.float32))
            return jnp.where(cnt >= kb, tt, t)

        t = lax.fori_loop(0, 32, tbody, jnp.zeros((1, 128), jnp.uint32),
                          unroll=True)
        c_gt = total((useg > t).astype(jnp.float32))
        r = kb - c_gt
        eq = useg == t

        def abody(j, a):
            bit = jnp.int32(14) - j
            at = a & ~(jnp.int32(1) << bit)
            cnt = total((eq & (idx <= at)).astype(jnp.float32))
            return jnp.where(cnt >= r, at, a)

        a = lax.fori_loop(0, 15, abody,
                          jnp.full((1, 128), SEG - 1, jnp.int32), unroll=True)
        ts = lax.bitcast_convert_type(t ^ jnp.uint32(0x80000000), jnp.int32)
        thr_ref[2 * b:2 * b + 1, :] = ts
        thr_ref[2 * b + 1:2 * b + 2, :] = a


def _apply_body(x_ref, c_ref, s_ref, thr_ref, wupf_ref, bupt_ref,
                xp_ref, sc_ref):
    i = pl.program_id(0)
    b = i // (NBLK // B)
    thr = thr_ref[...]

    def sel(row0):
        v = thr[row0 + 6:row0 + 7, 0:1]
        for bb in (2, 1, 0):
            v = jnp.where(b == bb, thr[row0 + 2 * bb:row0 + 2 * bb + 1, 0:1],
                          v)
        return v

    t = sel(0)
    a = sel(1)
    keys = _sortable_i32(s_ref[...])
    n_loc = lax.broadcasted_iota(jnp.int32, (R, 8), 0)
    o_idx = lax.broadcasted_iota(jnp.int32, (R, 8), 1)
    flat = ((i % (NBLK // B)) * R + n_loc) * 8 + o_idx
    mask = (keys > t) | ((keys == t) & (flat <= a))

    up_all = jnp.dot(x_ref[...].astype(jnp.bfloat16), wupf_ref[...],
                     preferred_element_type=jnp.float32) + bupt_ref[...]
    for o in range(8):
        mo = mask[:, o:o + 1]
        xp_ref[:, o, :] = jnp.where(
            mo, up_all[:, 256 * o:256 * (o + 1)], 0.0)

    c = c_ref[...]
    for o in range(8):
        dx, dy, dz = (o >> 2) & 1, (o >> 1) & 1, o & 1
        mo = mask[:, o:o + 1]
        newc = jnp.concatenate(
            [c[:, 0:1],
             c[:, 1:2] * 2 + dx,
             c[:, 2:3] * 2 + dy,
             c[:, 3:4] * 2 + dz], axis=1)
        sc_ref[:, o, :] = jnp.where(mo, newc, jnp.int32(-1))


@jax.jit
def kernel(x, coords, k, W_up, b_up, W1, b1, W2, b2):
    wupf = W_up.transpose(1, 0, 2).reshape(N_IN, 8 * N_OUT)
    bupt = jnp.tile(b_up, 8).reshape(1, 8 * N_OUT)
    w1 = W1.astype(jnp.bfloat16)
    b1r = b1.reshape(1, N_OUT)
    w2r = jnp.pad(W2, ((0, 0), (0, 127))).astype(jnp.bfloat16)
    b2r = b2.reshape(1, 1)
    wupf16 = wupf.astype(jnp.bfloat16)
    coords32 = coords.astype(jnp.int32)
    k32 = k.astype(jnp.int32)

    full = lambda shape: pl.BlockSpec(shape, lambda i: (0,) * len(shape))

    s_nat = pl.pallas_call(
        _scores_body,
        grid=(NBLK,),
        in_specs=[
            pl.BlockSpec((R, N_IN), lambda i: (i, 0)),
            full((N_IN, 8 * N_OUT)),
            full((1, 8 * N_OUT)),
            full((N_OUT, N_OUT)),
            full((1, N_OUT)),
            full((N_OUT, 128)),
            full((1, 1)),
        ],
        out_specs=pl.BlockSpec((R, 8), lambda i: (i, 0)),
        out_shape=jax.ShapeDtypeStruct((N_PTS, 8), jnp.float32),
        compiler_params=pltpu.CompilerParams(
            dimension_semantics=("parallel",)),
    )(x, wupf, bupt, w1, b1r, w2r, b2r)

    s2 = s_nat.reshape(N_PTS * 8 // 128, 128)
    thr = pl.pallas_call(
        _thresh_body,
        in_specs=[
            pl.BlockSpec(memory_space=pltpu.VMEM),
            pl.BlockSpec(memory_space=pltpu.SMEM),
        ],
        out_specs=pl.BlockSpec(memory_space=pltpu.VMEM),
        out_shape=jax.ShapeDtypeStruct((2 * B, 128), jnp.int32),
    )(s2, k32)

    xp_wide, sc_wide = pl.pallas_call(
        _apply_body,
        grid=(NBLK,),
        in_specs=[
            pl.BlockSpec((R, N_IN), lambda i: (i, 0)),
            pl.BlockSpec((R, 4), lambda i: (i, 0)),
            pl.BlockSpec((R, 8), lambda i: (i, 0)),
            full((2 * B, 128)),
            full((N_IN, 8 * N_OUT)),
            full((1, 8 * N_OUT)),
        ],
        out_specs=[
            pl.BlockSpec((R, 8, N_OUT), lambda i: (i, 0, 0)),
            pl.BlockSpec((R, 8, 4), lambda i: (i, 0, 0)),
        ],
        out_shape=[
            jax.ShapeDtypeStruct((N_PTS, 8, N_OUT), jnp.float32),
            jax.ShapeDtypeStruct((N_PTS, 8, 4), jnp.int32),
        ],
        compiler_params=pltpu.CompilerParams(
            dimension_semantics=("parallel",)),
    )(x, coords32, s_nat, thr, wupf16, bupt)

    x_pruned = xp_wide.reshape(N_PTS * 8, N_OUT)
    predictions = s_nat.reshape(N_PTS * 8, 1)
    sel_coords = sc_wide.reshape(N_PTS * 8, 4).astype(jnp.int64)
    import os as _os
    stage = _os.environ.get("KSTAGE", "full")
    if stage == "a":
        return (jnp.zeros((N_PTS * 8, N_OUT), jnp.float32), predictions,
                jnp.zeros((N_PTS * 8, 4), jnp.int64))
    if stage == "ab0":
        return (jnp.zeros((N_PTS * 8, N_OUT), jnp.float32) + thr[0, 0],
                predictions, jnp.zeros((N_PTS * 8, 4), jnp.int64))
    if stage == "noreshape":
        return (xp_wide, predictions, sel_coords)
    return x_pruned, predictions, sel_coords


# speedup vs baseline: 1.0398x; 1.0398x over previous
import functools
import jax
import jax.numpy as jnp
from jax import lax
from jax.experimental import pallas as pl
from jax.experimental.pallas import tpu as pltpu

N_PTS = 16384
N_IN = 256
N_OUT = 256
B = 4
R = 1024
NBLK = N_PTS // R
SEG = (N_PTS // B) * 8


def _sortable_i32(f32):
    i = lax.bitcast_convert_type(f32, jnp.int32)
    return i ^ (lax.shift_right_arithmetic(i, 31) & jnp.int32(0x7FFFFFFF))


def _scores_body(x_ref, wupf_ref, bupt_ref, w1_ref, b1_ref, w2_ref, b2_ref,
                 s_ref):
    xb = x_ref[...]
    up_all = jnp.dot(xb, wupf_ref[...], preferred_element_type=jnp.float32)
    up_all = up_all + bupt_ref[...]
    for o in range(8):
        up_o = up_all[:, 256 * o:256 * (o + 1)].astype(jnp.bfloat16)
        h = jnp.dot(up_o, w1_ref[...], preferred_element_type=jnp.float32)
        h = jnp.maximum(h + b1_ref[...], 0.0).astype(jnp.bfloat16)
        s = jnp.dot(h, w2_ref[...], preferred_element_type=jnp.float32)
        s_ref[:, o:o + 1] = s[:, 0:1] + b2_ref[0, 0]


def _thresh_body(s2_ref, k_ref, thr_ref):
    i32 = _sortable_i32(s2_ref[...])
    ukey = lax.bitcast_convert_type(i32, jnp.uint32) ^ jnp.uint32(0x80000000)
    rows = SEG // 128
    idx = (lax.broadcasted_iota(jnp.int32, (rows, 128), 0) * 128
           + lax.broadcasted_iota(jnp.int32, (rows, 128), 1))
    ones = jnp.ones((128, 128), jnp.float32)

    def total(pred_f32):
        part = jnp.sum(pred_f32, axis=0, keepdims=True)
        return jnp.dot(part, ones, preferred_element_type=jnp.float32)

    for b in range(B):
        useg = ukey[b * rows:(b + 1) * rows, :]
        kb = (k_ref[b]).astype(jnp.float32)

        def tbody(j, t):
            bit = jnp.uint32(31) - j.astype(jnp.uint32)
            tt = t | (jnp.uint32(1) << bit)
            cnt = total((useg >= tt).astype(jnp.float32))
            return jnp.where(cnt >= kb, tt, t)

        t = lax.fori_loop(0, 32, tbody, jnp.zeros((1, 128), jnp.uint32),
                          unroll=True)
        c_gt = total((useg > t).astype(jnp.float32))
        r = kb - c_gt
        eq = useg == t

        def abody(j, a):
            bit = jnp.int32(14) - j
            at = a & ~(jnp.int32(1) << bit)
            cnt = total((eq & (idx <= at)).astype(jnp.float32))
            return jnp.where(cnt >= r, at, a)

        a = lax.fori_loop(0, 15, abody,
                          jnp.full((1, 128), SEG - 1, jnp.int32), unroll=True)
        ts = lax.bitcast_convert_type(t ^ jnp.uint32(0x80000000), jnp.int32)
        thr_ref[2 * b:2 * b + 1, :] = ts
        thr_ref[2 * b + 1:2 * b + 2, :] = a


def _apply_body(x_ref, c_ref, s_ref, thr_ref, wupf_ref, bupt_ref,
                xp_ref, sc_ref):
    i = pl.program_id(0)
    b = i // (NBLK // B)
    thr = thr_ref[...]

    def sel(row0):
        v = thr[row0 + 6:row0 + 7, 0:1]
        for bb in (2, 1, 0):
            v = jnp.where(b == bb, thr[row0 + 2 * bb:row0 + 2 * bb + 1, 0:1],
                          v)
        return v

    t = sel(0)
    a = sel(1)
    keys = _sortable_i32(s_ref[...])
    n_loc = lax.broadcasted_iota(jnp.int32, (R, 8), 0)
    o_idx = lax.broadcasted_iota(jnp.int32, (R, 8), 1)
    flat = ((i % (NBLK // B)) * R + n_loc) * 8 + o_idx
    mask = (keys > t) | ((keys == t) & (flat <= a))

    up_all = jnp.dot(x_ref[...].astype(jnp.bfloat16), wupf_ref[...],
                     preferred_element_type=jnp.float32) + bupt_ref[...]
    for o in range(8):
        mo = mask[:, o:o + 1]
        xp_ref[:, o, :] = jnp.where(
            mo, up_all[:, 256 * o:256 * (o + 1)], 0.0)

    c = c_ref[...]
    for o in range(8):
        dx, dy, dz = (o >> 2) & 1, (o >> 1) & 1, o & 1
        mo = mask[:, o:o + 1]
        newc = jnp.concatenate(
            [c[:, 0:1],
             c[:, 1:2] * 2 + dx,
             c[:, 2:3] * 2 + dy,
             c[:, 3:4] * 2 + dz], axis=1)
        sc_ref[:, o, :] = jnp.where(mo, newc, jnp.int32(-1))


@jax.jit
def kernel(x, coords, k, W_up, b_up, W1, b1, W2, b2):
    wupf = W_up.transpose(1, 0, 2).reshape(N_IN, 8 * N_OUT)
    bupt = jnp.tile(b_up, 8).reshape(1, 8 * N_OUT)
    w1 = W1.astype(jnp.bfloat16)
    b1r = b1.reshape(1, N_OUT)
    w2r = jnp.pad(W2, ((0, 0), (0, 127))).astype(jnp.bfloat16)
    b2r = b2.reshape(1, 1)
    wupf16 = wupf.astype(jnp.bfloat16)
    coords32 = coords.astype(jnp.int32)
    k32 = k.astype(jnp.int32)

    full = lambda shape: pl.BlockSpec(shape, lambda i: (0,) * len(shape))

    s_nat = pl.pallas_call(
        _scores_body,
        grid=(NBLK,),
        in_specs=[
            pl.BlockSpec((R, N_IN), lambda i: (i, 0)),
            full((N_IN, 8 * N_OUT)),
            full((1, 8 * N_OUT)),
            full((N_OUT, N_OUT)),
            full((1, N_OUT)),
            full((N_OUT, 128)),
            full((1, 1)),
        ],
        out_specs=pl.BlockSpec((R, 8), lambda i: (i, 0)),
        out_shape=jax.ShapeDtypeStruct((N_PTS, 8), jnp.float32),
        compiler_params=pltpu.CompilerParams(
            dimension_semantics=("parallel",)),
    )(x, wupf, bupt, w1, b1r, w2r, b2r)

    s2 = s_nat.reshape(N_PTS * 8 // 128, 128)
    thr = pl.pallas_call(
        _thresh_body,
        in_specs=[
            pl.BlockSpec(memory_space=pltpu.VMEM),
            pl.BlockSpec(memory_space=pltpu.SMEM),
        ],
        out_specs=pl.BlockSpec(memory_space=pltpu.VMEM),
        out_shape=jax.ShapeDtypeStruct((2 * B, 128), jnp.int32),
    )(s2, k32)

    xp_wide, sc_wide = pl.pallas_call(
        _apply_body,
        grid=(NBLK,),
        in_specs=[
            pl.BlockSpec((R, N_IN), lambda i: (i, 0)),
            pl.BlockSpec((R, 4), lambda i: (i, 0)),
            pl.BlockSpec((R, 8), lambda i: (i, 0)),
            full((2 * B, 128)),
            full((N_IN, 8 * N_OUT)),
            full((1, 8 * N_OUT)),
        ],
        out_specs=[
            pl.BlockSpec((R, 8, N_OUT), lambda i: (i, 0, 0)),
            pl.BlockSpec((R, 8, 4), lambda i: (i, 0, 0)),
        ],
        out_shape=[
            jax.ShapeDtypeStruct((N_PTS, 8, N_OUT), jnp.float32),
            jax.ShapeDtypeStruct((N_PTS, 8, 4), jnp.int32),
        ],
        compiler_params=pltpu.CompilerParams(
            dimension_semantics=("parallel",)),
    )(x, coords32, s_nat, thr, wupf16, bupt)

    x_pruned = xp_wide.reshape(N_PTS * 8, N_OUT)
    predictions = s_nat.reshape(N_PTS * 8, 1)
    sel_coords = sc_wide.reshape(N_PTS * 8, 4).astype(jnp.int64)
    return x_pruned, predictions, sel_coords


# speedup vs baseline: 1.1231x; 1.0801x over previous
import functools
import jax
import jax.numpy as jnp
from jax import lax
from jax.experimental import pallas as pl
from jax.experimental.pallas import tpu as pltpu

N_PTS = 16384
N_IN = 256
N_OUT = 256
B = 4
R = 1024
NBLK = N_PTS // R
SEG = (N_PTS // B) * 8


def _sortable_i32(f32):
    i = lax.bitcast_convert_type(f32, jnp.int32)
    return i ^ (lax.shift_right_arithmetic(i, 31) & jnp.int32(0x7FFFFFFF))


def _scores_body(x_ref, wupf_ref, bupt_ref, w1_ref, b1_ref, w2_ref, b2_ref,
                 s_ref):
    xb = x_ref[...]
    up_all = jnp.dot(xb, wupf_ref[...], preferred_element_type=jnp.float32)
    up_all = up_all + bupt_ref[...]
    for o in range(8):
        up_o = up_all[:, 256 * o:256 * (o + 1)].astype(jnp.bfloat16)
        h = jnp.dot(up_o, w1_ref[...], preferred_element_type=jnp.float32)
        h = jnp.maximum(h + b1_ref[...], 0.0).astype(jnp.bfloat16)
        s = jnp.dot(h, w2_ref[...], preferred_element_type=jnp.float32)
        s_ref[:, o:o + 1] = s[:, 0:1] + b2_ref[0, 0]


def _thresh_body(s2_ref, k_ref, thr_ref):
    i32 = _sortable_i32(s2_ref[...])
    ukey = lax.bitcast_convert_type(i32, jnp.uint32) ^ jnp.uint32(0x80000000)
    rows = SEG // 128
    idx = (lax.broadcasted_iota(jnp.int32, (rows, 128), 0) * 128
           + lax.broadcasted_iota(jnp.int32, (rows, 128), 1))
    ones = jnp.ones((128, 128), jnp.float32)

    def total(pred_f32):
        part = jnp.sum(pred_f32, axis=0, keepdims=True)
        return jnp.dot(part, ones, preferred_element_type=jnp.float32)

    usegs = [ukey[b * rows:(b + 1) * rows, :] for b in range(B)]
    kbs = [(k_ref[b]).astype(jnp.float32) for b in range(B)]

    def tbody(j, ts):
        bit = jnp.uint32(31) - j.astype(jnp.uint32)
        out = []
        for b in range(B):
            tt = ts[b] | (jnp.uint32(1) << bit)
            cnt = total((usegs[b] >= tt).astype(jnp.float32))
            out.append(jnp.where(cnt >= kbs[b], tt, ts[b]))
        return tuple(out)

    ts = lax.fori_loop(0, 32, tbody,
                       tuple(jnp.zeros((1, 128), jnp.uint32)
                             for _ in range(B)), unroll=True)
    rs = [kbs[b] - total((usegs[b] > ts[b]).astype(jnp.float32))
          for b in range(B)]
    eqs = [usegs[b] == ts[b] for b in range(B)]

    def abody(j, aa):
        bit = jnp.int32(14) - j
        out = []
        for b in range(B):
            at = aa[b] & ~(jnp.int32(1) << bit)
            cnt = total((eqs[b] & (idx <= at)).astype(jnp.float32))
            out.append(jnp.where(cnt >= rs[b], at, aa[b]))
        return tuple(out)

    aa = lax.fori_loop(0, 15, abody,
                       tuple(jnp.full((1, 128), SEG - 1, jnp.int32)
                             for _ in range(B)), unroll=True)
    for b in range(B):
        ts_i = lax.bitcast_convert_type(ts[b] ^ jnp.uint32(0x80000000),
                                        jnp.int32)
        thr_ref[2 * b:2 * b + 1, :] = ts_i
        thr_ref[2 * b + 1:2 * b + 2, :] = aa[b]


def _apply_body(x_ref, c_ref, s_ref, thr_ref, wupf_ref, bupt_ref,
                xp_ref, sc_ref):
    i = pl.program_id(0)
    b = i // (NBLK // B)
    thr = thr_ref[...]

    def sel(row0):
        v = thr[row0 + 6:row0 + 7, 0:1]
        for bb in (2, 1, 0):
            v = jnp.where(b == bb, thr[row0 + 2 * bb:row0 + 2 * bb + 1, 0:1],
                          v)
        return v

    t = sel(0)
    a = sel(1)
    keys = _sortable_i32(s_ref[...])
    n_loc = lax.broadcasted_iota(jnp.int32, (R, 8), 0)
    o_idx = lax.broadcasted_iota(jnp.int32, (R, 8), 1)
    flat = ((i % (NBLK // B)) * R + n_loc) * 8 + o_idx
    mask = (keys > t) | ((keys == t) & (flat <= a))

    up_all = jnp.dot(x_ref[...].astype(jnp.bfloat16), wupf_ref[...],
                     preferred_element_type=jnp.float32) + bupt_ref[...]
    for o in range(8):
        mo = mask[:, o:o + 1]
        xp_ref[:, o, :] = jnp.where(
            mo, up_all[:, 256 * o:256 * (o + 1)], 0.0)

    c = c_ref[...]
    for o in range(8):
        dx, dy, dz = (o >> 2) & 1, (o >> 1) & 1, o & 1
        mo = mask[:, o:o + 1]
        newc = jnp.concatenate(
            [c[:, 0:1],
             c[:, 1:2] * 2 + dx,
             c[:, 2:3] * 2 + dy,
             c[:, 3:4] * 2 + dz], axis=1)
        sc_ref[:, o, :] = jnp.where(mo, newc, jnp.int32(-1))


@jax.jit
def kernel(x, coords, k, W_up, b_up, W1, b1, W2, b2):
    wupf = W_up.transpose(1, 0, 2).reshape(N_IN, 8 * N_OUT)
    bupt = jnp.tile(b_up, 8).reshape(1, 8 * N_OUT)
    w1 = W1.astype(jnp.bfloat16)
    b1r = b1.reshape(1, N_OUT)
    w2r = jnp.pad(W2, ((0, 0), (0, 127))).astype(jnp.bfloat16)
    b2r = b2.reshape(1, 1)
    wupf16 = wupf.astype(jnp.bfloat16)
    coords32 = coords.astype(jnp.int32)
    k32 = k.astype(jnp.int32)

    full = lambda shape: pl.BlockSpec(shape, lambda i: (0,) * len(shape))

    s_nat = pl.pallas_call(
        _scores_body,
        grid=(NBLK,),
        in_specs=[
            pl.BlockSpec((R, N_IN), lambda i: (i, 0)),
            full((N_IN, 8 * N_OUT)),
            full((1, 8 * N_OUT)),
            full((N_OUT, N_OUT)),
            full((1, N_OUT)),
            full((N_OUT, 128)),
            full((1, 1)),
        ],
        out_specs=pl.BlockSpec((R, 8), lambda i: (i, 0)),
        out_shape=jax.ShapeDtypeStruct((N_PTS, 8), jnp.float32),
        compiler_params=pltpu.CompilerParams(
            dimension_semantics=("parallel",)),
    )(x, wupf, bupt, w1, b1r, w2r, b2r)

    s2 = s_nat.reshape(N_PTS * 8 // 128, 128)
    thr = pl.pallas_call(
        _thresh_body,
        in_specs=[
            pl.BlockSpec(memory_space=pltpu.VMEM),
            pl.BlockSpec(memory_space=pltpu.SMEM),
        ],
        out_specs=pl.BlockSpec(memory_space=pltpu.VMEM),
        out_shape=jax.ShapeDtypeStruct((2 * B, 128), jnp.int32),
    )(s2, k32)

    xp_wide, sc_wide = pl.pallas_call(
        _apply_body,
        grid=(NBLK,),
        in_specs=[
            pl.BlockSpec((R, N_IN), lambda i: (i, 0)),
            pl.BlockSpec((R, 4), lambda i: (i, 0)),
            pl.BlockSpec((R, 8), lambda i: (i, 0)),
            full((2 * B, 128)),
            full((N_IN, 8 * N_OUT)),
            full((1, 8 * N_OUT)),
        ],
        out_specs=[
            pl.BlockSpec((R, 8, N_OUT), lambda i: (i, 0, 0)),
            pl.BlockSpec((R, 8, 4), lambda i: (i, 0, 0)),
        ],
        out_shape=[
            jax.ShapeDtypeStruct((N_PTS, 8, N_OUT), jnp.float32),
            jax.ShapeDtypeStruct((N_PTS, 8, 4), jnp.int32),
        ],
        compiler_params=pltpu.CompilerParams(
            dimension_semantics=("parallel",)),
    )(x, coords32, s_nat, thr, wupf16, bupt)

    x_pruned = xp_wide.reshape(N_PTS * 8, N_OUT)
    predictions = s_nat.reshape(N_PTS * 8, 1)
    sel_coords = sc_wide.reshape(N_PTS * 8, 4).astype(jnp.int64)
    return x_pruned, predictions, sel_coords
